# baseline (speedup 1.0000x reference)
import functools

import jax
import jax.numpy as jnp
from jax import lax
from jax.experimental import pallas as pl
from jax.experimental.pallas import tpu as pltpu

BATCH = 1024
N_ACT = 100000

ROWS = 8
COLT = 2048
CHUNK = 512
CBLOCKS = (N_ACT + COLT - 1) // COLT

_KS0 = 0
_KS1 = 42
_KS2 = 42 ^ 0x1BD11BDA
_ROT_A = (13, 15, 26, 6)
_ROT_B = (17, 29, 16, 24)
_INJ = ((_KS1, _KS2 + 1), (_KS2, _KS0 + 2), (_KS0, _KS1 + 3),
        (_KS1, _KS2 + 4), (_KS2, _KS0 + 5))


def _rotl(x, d):
    return lax.shift_left(x, jnp.int32(d)) | lax.shift_right_logical(
        x, jnp.int32(32 - d))


def _threefry_bits(cnt):
    x1 = cnt + jnp.int32(_KS1)
    x0 = x1
    x1 = _rotl(x1, _ROT_A[0]) ^ x0
    first = True
    for g in range(5):
        rots = _ROT_A if g % 2 == 0 else _ROT_B
        for r in rots:
            if first:
                first = False
                continue
            x0 = x0 + x1
            x1 = _rotl(x1, r) ^ x0
        a, b = _INJ[g]
        x0 = x0 + jnp.int32(a)
        x1 = x1 + jnp.int32(b)
    return x0 ^ x1


def _body(mask_ref, act_ref, lp_ref, bv_ref, bc_ref, cnt_ref):
    r = pl.program_id(0)
    c = pl.program_id(1)
    row0 = r * ROWS
    col0 = c * COLT

    @pl.when(c == 0)
    def _init():
        bv_ref[...] = jnp.full((ROWS, 1), -2, jnp.int32)
        bc_ref[...] = jnp.zeros((ROWS, 1), jnp.int32)
        cnt_ref[...] = jnp.zeros((ROWS, 1), jnp.int32)

    rowbase = (row0 + lax.broadcasted_iota(jnp.int32, (ROWS, CHUNK), 0)) \
        * jnp.int32(N_ACT)

    def chunk(i, _):
        col = col0 + i * CHUNK + lax.broadcasted_iota(
            jnp.int32, (ROWS, CHUNK), 1)
        m = mask_ref[:, pl.ds(i * CHUNK, CHUNK)]
        valid = m & (col < N_ACT)
        bits = lax.shift_right_logical(_threefry_bits(rowbase + col),
                                       jnp.int32(9))
        v = jnp.where(valid, bits, jnp.int32(-1))
        mx = jnp.max(v, axis=1, keepdims=True)
        idx = jnp.min(jnp.where(v == mx, col, jnp.int32(2**30)),
                      axis=1, keepdims=True)
        upd = mx > bv_ref[...]
        bc_ref[...] = jnp.where(upd, idx, bc_ref[...])
        bv_ref[...] = jnp.where(upd, mx, bv_ref[...])
        cnt_ref[...] = cnt_ref[...] + jnp.sum(
            valid.astype(jnp.int32), axis=1, keepdims=True)
        return 0

    lax.fori_loop(0, COLT // CHUNK, chunk, 0, unroll=True)

    @pl.when(c == CBLOCKS - 1)
    def _fin():
        act_ref[...] = bc_ref[...]
        lp_ref[...] = -jnp.log(cnt_ref[...].astype(jnp.float32))


@functools.partial(jax.jit, static_argnames=())
def _sample(mask):
    act, lp = pl.pallas_call(
        _body,
        grid=(BATCH // ROWS, CBLOCKS),
        in_specs=[pl.BlockSpec((ROWS, COLT), lambda r, c: (r, c))],
        out_specs=[pl.BlockSpec((ROWS, 1), lambda r, c: (r, 0)),
                   pl.BlockSpec((ROWS, 1), lambda r, c: (r, 0))],
        out_shape=[jax.ShapeDtypeStruct((BATCH, 1), jnp.int32),
                   jax.ShapeDtypeStruct((BATCH, 1), jnp.float32)],
        scratch_shapes=[pltpu.VMEM((ROWS, 1), jnp.int32),
                        pltpu.VMEM((ROWS, 1), jnp.int32),
                        pltpu.VMEM((ROWS, 1), jnp.int32)],
        compiler_params=pltpu.CompilerParams(
            dimension_semantics=("arbitrary", "arbitrary")),
    )(mask)
    return act[:, 0], lp[:, 0]


def kernel(action_mask, fc_w, fc_b):
    del fc_w, fc_b
    return _sample(action_mask.astype(jnp.bool_))


# speedup vs baseline: 1.1926x; 1.1926x over previous
import functools

import jax
import jax.numpy as jnp
from jax import lax
from jax.experimental import pallas as pl
from jax.experimental.pallas import tpu as pltpu

BATCH = 1024
N_ACT = 100000

ROWS = 8
COLT = 2048
CHUNK = 2048
CBLOCKS = (N_ACT + COLT - 1) // COLT

_KS0 = 0
_KS1 = 42
_KS2 = 42 ^ 0x1BD11BDA
_ROT_A = (13, 15, 26, 6)
_ROT_B = (17, 29, 16, 24)
_INJ = ((_KS1, _KS2 + 1), (_KS2, _KS0 + 2), (_KS0, _KS1 + 3),
        (_KS1, _KS2 + 4), (_KS2, _KS0 + 5))


def _rotl(x, d):
    return lax.shift_left(x, jnp.int32(d)) | lax.shift_right_logical(
        x, jnp.int32(32 - d))


def _threefry_bits(cnt):
    x1 = cnt + jnp.int32(_KS1)
    x0 = x1
    x1 = _rotl(x1, _ROT_A[0]) ^ x0
    first = True
    for g in range(5):
        rots = _ROT_A if g % 2 == 0 else _ROT_B
        for r in rots:
            if first:
                first = False
                continue
            x0 = x0 + x1
            x1 = _rotl(x1, r) ^ x0
        a, b = _INJ[g]
        x0 = x0 + jnp.int32(a)
        x1 = x1 + jnp.int32(b)
    return x0 ^ x1


def _body(mask_ref, act_ref, lp_ref, key_acc, cnt_acc):
    r = pl.program_id(0)
    c = pl.program_id(1)
    row0 = r * ROWS

    @pl.when(c == 0)
    def _init():
        key_acc[...] = jnp.full((ROWS, COLT), -1, jnp.int32)
        cnt_acc[...] = jnp.zeros((ROWS, COLT), jnp.int32)

    lane = lax.broadcasted_iota(jnp.int32, (ROWS, COLT), 1)
    col = c * COLT + lane
    rowbase = (row0 + lax.broadcasted_iota(jnp.int32, (ROWS, COLT), 0)) \
        * jnp.int32(N_ACT)
    m = mask_ref[...]
    valid = m & (col < N_ACT)
    bits = _threefry_bits(rowbase + col)
    key = (lax.shift_right_logical(bits, jnp.int32(3))
           & jnp.int32(0x7FFFFFC0)) | (jnp.int32(CBLOCKS - 1) - c)
    v = jnp.where(valid, key, jnp.int32(-1))
    key_acc[...] = jnp.maximum(key_acc[...], v)
    cnt_acc[...] = cnt_acc[...] + valid.astype(jnp.int32)

    @pl.when(c == CBLOCKS - 1)
    def _fin():
        keys = key_acc[...]
        bb = lax.shift_right_arithmetic(keys, jnp.int32(6))
        strip = jnp.int32(CBLOCKS - 1) - (keys & jnp.int32(63))
        gcol = strip * jnp.int32(COLT) + lane
        mx = jnp.max(bb, axis=1, keepdims=True)
        act_ref[...] = jnp.min(
            jnp.where(bb == mx, gcol, jnp.int32(2**30)),
            axis=1, keepdims=True)
        cnt = jnp.sum(cnt_acc[...], axis=1, keepdims=True)
        lp_ref[...] = -jnp.log(cnt.astype(jnp.float32))


@functools.partial(jax.jit, static_argnames=())
def _sample(mask):
    act, lp = pl.pallas_call(
        _body,
        grid=(BATCH // ROWS, CBLOCKS),
        in_specs=[pl.BlockSpec((ROWS, COLT), lambda r, c: (r, c))],
        out_specs=[pl.BlockSpec((ROWS, 1), lambda r, c: (r, 0)),
                   pl.BlockSpec((ROWS, 1), lambda r, c: (r, 0))],
        out_shape=[jax.ShapeDtypeStruct((BATCH, 1), jnp.int32),
                   jax.ShapeDtypeStruct((BATCH, 1), jnp.float32)],
        scratch_shapes=[pltpu.VMEM((ROWS, COLT), jnp.int32),
                        pltpu.VMEM((ROWS, COLT), jnp.int32)],
        compiler_params=pltpu.CompilerParams(
            dimension_semantics=("arbitrary", "arbitrary")),
    )(mask)
    return act[:, 0], lp[:, 0]


def kernel(action_mask, fc_w, fc_b):
    del fc_w, fc_b
    return _sample(action_mask.astype(jnp.bool_))


# speedup vs baseline: 2.0319x; 1.7038x over previous
import functools

import jax
import jax.numpy as jnp
from jax import lax
from jax.experimental import pallas as pl
from jax.experimental.pallas import tpu as pltpu

BATCH = 1024
N_ACT = 100000

ROWS = 32
COLT = 8192
RSUB = 8
CSUB = 2048
CBLOCKS = (N_ACT + COLT - 1) // COLT
SB = (CBLOCKS - 1).bit_length()
KEYMASK = ((2**23 - 1) << SB) & 0x7FFFFFFF

_KS0 = 0
_KS1 = 42
_KS2 = 42 ^ 0x1BD11BDA
_ROT_A = (13, 15, 26, 6)
_ROT_B = (17, 29, 16, 24)
_INJ = ((_KS1, _KS2 + 1), (_KS2, _KS0 + 2), (_KS0, _KS1 + 3),
        (_KS1, _KS2 + 4), (_KS2, _KS0 + 5))


def _rotl(x, d):
    return lax.shift_left(x, jnp.int32(d)) | lax.shift_right_logical(
        x, jnp.int32(32 - d))


def _threefry_bits(cnt):
    x1 = cnt + jnp.int32(_KS1)
    x0 = x1
    x1 = _rotl(x1, _ROT_A[0]) ^ x0
    first = True
    for g in range(5):
        rots = _ROT_A if g % 2 == 0 else _ROT_B
        for r in rots:
            if first:
                first = False
                continue
            x0 = x0 + x1
            x1 = _rotl(x1, r) ^ x0
        a, b = _INJ[g]
        x0 = x0 + jnp.int32(a)
        x1 = x1 + jnp.int32(b)
    return x0 ^ x1


def _body(mask_ref, act_ref, lp_ref, key_acc, cnt_acc):
    r = pl.program_id(0)
    c = pl.program_id(1)

    @pl.when(c == 0)
    def _init():
        key_acc[...] = jnp.full((ROWS, COLT), -1, jnp.int32)
        cnt_acc[...] = jnp.zeros((ROWS, COLT), jnp.int32)

    lane = lax.broadcasted_iota(jnp.int32, (RSUB, CSUB), 1)
    iota0 = lax.broadcasted_iota(jnp.int32, (RSUB, CSUB), 0)
    revstrip = jnp.int32(CBLOCKS - 1) - c

    def chunk(k, _):
        ri = pl.multiple_of((k // (COLT // CSUB)) * RSUB, RSUB)
        ci = pl.multiple_of((k % (COLT // CSUB)) * CSUB, 256)
        col = c * COLT + ci + lane
        rowbase = (r * ROWS + ri + iota0) * jnp.int32(N_ACT)
        m = mask_ref[pl.ds(ri, RSUB), pl.ds(ci, CSUB)]
        valid = m & (col < N_ACT)
        bits = _threefry_bits(rowbase + col)
        key = (lax.shift_right_logical(bits, jnp.int32(9 - SB))
               & jnp.int32(KEYMASK)) | revstrip
        v = jnp.where(valid, key, jnp.int32(-1))
        ka = key_acc[pl.ds(ri, RSUB), pl.ds(ci, CSUB)]
        key_acc[pl.ds(ri, RSUB), pl.ds(ci, CSUB)] = jnp.maximum(ka, v)
        ca = cnt_acc[pl.ds(ri, RSUB), pl.ds(ci, CSUB)]
        cnt_acc[pl.ds(ri, RSUB), pl.ds(ci, CSUB)] = ca + valid.astype(jnp.int32)
        return 0

    lax.fori_loop(0, (ROWS // RSUB) * (COLT // CSUB), chunk, 0)

    @pl.when(c == CBLOCKS - 1)
    def _fin():
        lane_f = lax.broadcasted_iota(jnp.int32, (RSUB, COLT), 1)
        for ri in range(ROWS // RSUB):
            keys = key_acc[pl.ds(ri * RSUB, RSUB), :]
            bb = lax.shift_right_arithmetic(keys, jnp.int32(SB))
            strip = jnp.int32(CBLOCKS - 1) - (keys & jnp.int32(2**SB - 1))
            gcol = strip * jnp.int32(COLT) + lane_f
            mx = jnp.max(bb, axis=1, keepdims=True)
            act_ref[pl.ds(ri * RSUB, RSUB), :] = jnp.min(
                jnp.where(bb == mx, gcol, jnp.int32(2**30)),
                axis=1, keepdims=True)
            cnt = jnp.sum(cnt_acc[pl.ds(ri * RSUB, RSUB), :],
                          axis=1, keepdims=True)
            lp_ref[pl.ds(ri * RSUB, RSUB), :] = -jnp.log(
                cnt.astype(jnp.float32))


@jax.jit
def _sample(mask):
    act, lp = pl.pallas_call(
        _body,
        grid=(BATCH // ROWS, CBLOCKS),
        in_specs=[pl.BlockSpec((ROWS, COLT), lambda r, c: (r, c))],
        out_specs=[pl.BlockSpec((ROWS, 1), lambda r, c: (r, 0)),
                   pl.BlockSpec((ROWS, 1), lambda r, c: (r, 0))],
        out_shape=[jax.ShapeDtypeStruct((BATCH, 1), jnp.int32),
                   jax.ShapeDtypeStruct((BATCH, 1), jnp.float32)],
        scratch_shapes=[pltpu.VMEM((ROWS, COLT), jnp.int32),
                        pltpu.VMEM((ROWS, COLT), jnp.int32)],
        compiler_params=pltpu.CompilerParams(
            dimension_semantics=("arbitrary", "arbitrary")),
    )(mask)
    return act[:, 0], lp[:, 0]


def kernel(action_mask, fc_w, fc_b):
    del fc_w, fc_b
    return _sample(action_mask.astype(jnp.bool_))


# speedup vs baseline: 2.0684x; 1.0180x over previous
import functools

import jax
import jax.numpy as jnp
from jax import lax
from jax.experimental import pallas as pl
from jax.experimental.pallas import tpu as pltpu

BATCH = 1024
N_ACT = 100000

ROWS = 32
COLT = 8192
RSUB = 8
CSUB = 2048
CBLOCKS = (N_ACT + COLT - 1) // COLT
SB = (CBLOCKS - 1).bit_length()
KEYMASK = ((2**23 - 1) << SB) & 0x7FFFFFFF

_KS0 = 0
_KS1 = 42
_KS2 = 42 ^ 0x1BD11BDA
_ROT_A = (13, 15, 26, 6)
_ROT_B = (17, 29, 16, 24)
_INJ = ((_KS1, _KS2 + 1), (_KS2, _KS0 + 2), (_KS0, _KS1 + 3),
        (_KS1, _KS2 + 4), (_KS2, _KS0 + 5))


def _rotl(x, d):
    return lax.shift_left(x, jnp.int32(d)) | lax.shift_right_logical(
        x, jnp.int32(32 - d))


def _threefry_bits(x1):
    x0 = x1
    x1 = _rotl(x1, _ROT_A[0]) ^ x0
    first = True
    for g in range(5):
        rots = _ROT_A if g % 2 == 0 else _ROT_B
        for r in rots:
            if first:
                first = False
                continue
            x0 = x0 + x1
            x1 = _rotl(x1, r) ^ x0
        a, b = _INJ[g]
        x0 = x0 + jnp.int32(a)
        x1 = x1 + jnp.int32(b)
    return x0 ^ x1


def _body(mask_ref, act_ref, lp_ref, key_acc, cnt_acc):
    r = pl.program_id(0)
    c = pl.program_id(1)

    @pl.when(c == 0)
    def _init():
        key_acc[...] = jnp.full((ROWS, COLT), -1, jnp.int32)
        cnt_acc[...] = jnp.zeros((ROWS, COLT), jnp.int32)

    lane = lax.broadcasted_iota(jnp.int32, (RSUB, CSUB), 1)
    iota0 = lax.broadcasted_iota(jnp.int32, (RSUB, CSUB), 0)
    revstrip = jnp.int32(CBLOCKS - 1) - c
    base2d = iota0 * jnp.int32(N_ACT) + lane + jnp.int32(_KS1)
    scal0 = r * jnp.int32(ROWS * N_ACT) + c * jnp.int32(COLT)
    nchunk = (ROWS // RSUB) * (COLT // CSUB)

    def make_chunk(guarded):
        def chunk(k, _):
            ri = pl.multiple_of((k // (COLT // CSUB)) * RSUB, RSUB)
            ci = pl.multiple_of((k % (COLT // CSUB)) * CSUB, 256)
            m = mask_ref[pl.ds(ri, RSUB), pl.ds(ci, CSUB)]
            if guarded:
                valid = m & (lane < (jnp.int32(N_ACT) - c * jnp.int32(COLT)
                                     - ci))
            else:
                valid = m
            bits = _threefry_bits(base2d + (scal0 + ri * jnp.int32(N_ACT)
                                            + ci))
            key = (lax.shift_right_logical(bits, jnp.int32(9 - SB))
                   & jnp.int32(KEYMASK)) | revstrip
            v = jnp.where(valid, key, jnp.int32(-1))
            ka = key_acc[pl.ds(ri, RSUB), pl.ds(ci, CSUB)]
            key_acc[pl.ds(ri, RSUB), pl.ds(ci, CSUB)] = jnp.maximum(ka, v)
            ca = cnt_acc[pl.ds(ri, RSUB), pl.ds(ci, CSUB)]
            cnt_acc[pl.ds(ri, RSUB), pl.ds(ci, CSUB)] = \
                ca + valid.astype(jnp.int32)
            return 0
        return chunk

    @pl.when(c < CBLOCKS - 1)
    def _main():
        lax.fori_loop(0, nchunk, make_chunk(False), 0)

    @pl.when(c == CBLOCKS - 1)
    def _tail():
        lax.fori_loop(0, nchunk, make_chunk(True), 0)

    @pl.when(c == CBLOCKS - 1)
    def _fin():
        lane_f = lax.broadcasted_iota(jnp.int32, (RSUB, COLT), 1)
        for ri in range(ROWS // RSUB):
            keys = key_acc[pl.ds(ri * RSUB, RSUB), :]
            bb = lax.shift_right_arithmetic(keys, jnp.int32(SB))
            strip = jnp.int32(CBLOCKS - 1) - (keys & jnp.int32(2**SB - 1))
            gcol = strip * jnp.int32(COLT) + lane_f
            mx = jnp.max(bb, axis=1, keepdims=True)
            act_ref[pl.ds(ri * RSUB, RSUB), :] = jnp.min(
                jnp.where(bb == mx, gcol, jnp.int32(2**30)),
                axis=1, keepdims=True)
            cnt = jnp.sum(cnt_acc[pl.ds(ri * RSUB, RSUB), :],
                          axis=1, keepdims=True)
            lp_ref[pl.ds(ri * RSUB, RSUB), :] = -jnp.log(
                cnt.astype(jnp.float32))


@jax.jit
def _sample(mask):
    act, lp = pl.pallas_call(
        _body,
        grid=(BATCH // ROWS, CBLOCKS),
        in_specs=[pl.BlockSpec((ROWS, COLT), lambda r, c: (r, c))],
        out_specs=[pl.BlockSpec((ROWS, 1), lambda r, c: (r, 0)),
                   pl.BlockSpec((ROWS, 1), lambda r, c: (r, 0))],
        out_shape=[jax.ShapeDtypeStruct((BATCH, 1), jnp.int32),
                   jax.ShapeDtypeStruct((BATCH, 1), jnp.float32)],
        scratch_shapes=[pltpu.VMEM((ROWS, COLT), jnp.int32),
                        pltpu.VMEM((ROWS, COLT), jnp.int32)],
        compiler_params=pltpu.CompilerParams(
            dimension_semantics=("arbitrary", "arbitrary")),
    )(mask)
    return act[:, 0], lp[:, 0]


def kernel(action_mask, fc_w, fc_b):
    del fc_w, fc_b
    return _sample(action_mask.astype(jnp.bool_))


# speedup vs baseline: 2.1744x; 1.0512x over previous
import functools

import jax
import jax.numpy as jnp
from jax import lax
from jax.experimental import pallas as pl
from jax.experimental.pallas import tpu as pltpu

BATCH = 1024
N_ACT = 100000

ROWS = 32
COLT = 8192
RSUB = 8
CSUB = 2048
CBLOCKS = (N_ACT + COLT - 1) // COLT
SB = (CBLOCKS - 1).bit_length()
KEYMASK = ((2**23 - 1) << SB) & 0x7FFFFFFF

_KS0 = 0
_KS1 = 42
_KS2 = 42 ^ 0x1BD11BDA
_ROT_A = (13, 15, 26, 6)
_ROT_B = (17, 29, 16, 24)
_INJ = ((_KS1, _KS2 + 1), (_KS2, _KS0 + 2), (_KS0, _KS1 + 3),
        (_KS1, _KS2 + 4), (_KS2, _KS0 + 5))


def _rotl(x, d):
    return lax.shift_left(x, jnp.int32(d)) | lax.shift_right_logical(
        x, jnp.int32(32 - d))


def _threefry_bits(x1):
    x0 = x1
    x1 = _rotl(x1, _ROT_A[0]) ^ x0
    first = True
    for g in range(5):
        rots = _ROT_A if g % 2 == 0 else _ROT_B
        for r in rots:
            if first:
                first = False
                continue
            x0 = x0 + x1
            x1 = _rotl(x1, r) ^ x0
        a, b = _INJ[g]
        x0 = x0 + jnp.int32(a)
        x1 = x1 + jnp.int32(b)
    return x0 ^ x1


def _body(mask_ref, act_ref, lp_ref, key_acc, cnt_acc):
    r = pl.program_id(0)
    c = pl.program_id(1)

    @pl.when(c == 0)
    def _init():
        key_acc[...] = jnp.full((ROWS, COLT), -1, jnp.int32)
        cnt_acc[...] = jnp.zeros((ROWS, COLT), jnp.int32)

    lane = lax.broadcasted_iota(jnp.int32, (RSUB, CSUB), 1)
    iota0 = lax.broadcasted_iota(jnp.int32, (RSUB, CSUB), 0)
    revstrip = jnp.int32(CBLOCKS - 1) - c
    base2d = iota0 * jnp.int32(N_ACT) + lane + jnp.int32(_KS1)
    scal0 = r * jnp.int32(ROWS * N_ACT) + c * jnp.int32(COLT)
    nchunk = (ROWS // RSUB) * (COLT // CSUB)

    def make_chunk(guarded):
        def chunk(k, _):
            ri = pl.multiple_of((k // (COLT // CSUB)) * RSUB, RSUB)
            ci = pl.multiple_of((k % (COLT // CSUB)) * CSUB, 256)
            m = mask_ref[pl.ds(ri, RSUB), pl.ds(ci, CSUB)]
            if guarded:
                valid = m & (lane < (jnp.int32(N_ACT) - c * jnp.int32(COLT)
                                     - ci))
            else:
                valid = m
            bits = _threefry_bits(base2d + (scal0 + ri * jnp.int32(N_ACT)
                                            + ci))
            key = (lax.shift_right_logical(bits, jnp.int32(9 - SB))
                   & jnp.int32(KEYMASK)) | revstrip
            v = jnp.where(valid, key, jnp.int32(-1))
            ka = key_acc[pl.ds(ri, RSUB), pl.ds(ci, CSUB)]
            key_acc[pl.ds(ri, RSUB), pl.ds(ci, CSUB)] = jnp.maximum(ka, v)
            ca = cnt_acc[pl.ds(ri, RSUB), pl.ds(ci, CSUB)]
            cnt_acc[pl.ds(ri, RSUB), pl.ds(ci, CSUB)] = \
                ca + valid.astype(jnp.int32)
            return 0
        return chunk

    @pl.when(c < CBLOCKS - 1)
    def _main():
        lax.fori_loop(0, nchunk, make_chunk(False), 0, unroll=4)

    @pl.when(c == CBLOCKS - 1)
    def _tail():
        lax.fori_loop(0, nchunk, make_chunk(True), 0, unroll=4)

    @pl.when(c == CBLOCKS - 1)
    def _fin():
        lane_f = lax.broadcasted_iota(jnp.int32, (RSUB, COLT), 1)
        for ri in range(ROWS // RSUB):
            keys = key_acc[pl.ds(ri * RSUB, RSUB), :]
            bb = lax.shift_right_arithmetic(keys, jnp.int32(SB))
            strip = jnp.int32(CBLOCKS - 1) - (keys & jnp.int32(2**SB - 1))
            gcol = strip * jnp.int32(COLT) + lane_f
            mx = jnp.max(bb, axis=1, keepdims=True)
            act_ref[pl.ds(ri * RSUB, RSUB), :] = jnp.min(
                jnp.where(bb == mx, gcol, jnp.int32(2**30)),
                axis=1, keepdims=True)
            cnt = jnp.sum(cnt_acc[pl.ds(ri * RSUB, RSUB), :],
                          axis=1, keepdims=True)
            lp_ref[pl.ds(ri * RSUB, RSUB), :] = -jnp.log(
                cnt.astype(jnp.float32))


@jax.jit
def _sample(mask):
    act, lp = pl.pallas_call(
        _body,
        grid=(BATCH // ROWS, CBLOCKS),
        in_specs=[pl.BlockSpec((ROWS, COLT), lambda r, c: (r, c))],
        out_specs=[pl.BlockSpec((ROWS, 1), lambda r, c: (r, 0)),
                   pl.BlockSpec((ROWS, 1), lambda r, c: (r, 0))],
        out_shape=[jax.ShapeDtypeStruct((BATCH, 1), jnp.int32),
                   jax.ShapeDtypeStruct((BATCH, 1), jnp.float32)],
        scratch_shapes=[pltpu.VMEM((ROWS, COLT), jnp.int32),
                        pltpu.VMEM((ROWS, COLT), jnp.int32)],
        compiler_params=pltpu.CompilerParams(
            dimension_semantics=("arbitrary", "arbitrary")),
    )(mask)
    return act[:, 0], lp[:, 0]


def kernel(action_mask, fc_w, fc_b):
    del fc_w, fc_b
    return _sample(action_mask.astype(jnp.bool_))


# speedup vs baseline: 2.2668x; 1.0425x over previous
import functools

import jax
import jax.numpy as jnp
from jax import lax
from jax.experimental import pallas as pl
from jax.experimental.pallas import tpu as pltpu

BATCH = 1024
N_ACT = 100000

ROWS = 128
COLT = 4096
RSUB = 8
CSUB = 2048
CBLOCKS = (N_ACT + COLT - 1) // COLT
SB = (CBLOCKS - 1).bit_length()
KEYMASK = ((2**23 - 1) << SB) & 0x7FFFFFFF

_KS0 = 0
_KS1 = 42
_KS2 = 42 ^ 0x1BD11BDA
_ROT_A = (13, 15, 26, 6)
_ROT_B = (17, 29, 16, 24)
_INJ = ((_KS1, _KS2 + 1), (_KS2, _KS0 + 2), (_KS0, _KS1 + 3),
        (_KS1, _KS2 + 4), (_KS2, _KS0 + 5))


def _rotl(x, d):
    return lax.shift_left(x, jnp.int32(d)) | lax.shift_right_logical(
        x, jnp.int32(32 - d))


def _threefry_bits(x1):
    x0 = x1
    x1 = _rotl(x1, _ROT_A[0]) ^ x0
    first = True
    for g in range(5):
        rots = _ROT_A if g % 2 == 0 else _ROT_B
        for r in rots:
            if first:
                first = False
                continue
            x0 = x0 + x1
            x1 = _rotl(x1, r) ^ x0
        a, b = _INJ[g]
        x0 = x0 + jnp.int32(a)
        x1 = x1 + jnp.int32(b)
    return x0 ^ x1


def _body(mask_ref, act_ref, lp_ref, key_acc, cnt_acc):
    r = pl.program_id(0)
    c = pl.program_id(1)

    @pl.when(c == 0)
    def _init():
        key_acc[...] = jnp.full((ROWS, COLT), -1, jnp.int32)
        cnt_acc[...] = jnp.zeros((ROWS, COLT), jnp.int32)

    lane = lax.broadcasted_iota(jnp.int32, (RSUB, CSUB), 1)
    iota0 = lax.broadcasted_iota(jnp.int32, (RSUB, CSUB), 0)
    revstrip = jnp.int32(CBLOCKS - 1) - c
    base2d = iota0 * jnp.int32(N_ACT) + lane + jnp.int32(_KS1)
    scal0 = r * jnp.int32(ROWS * N_ACT) + c * jnp.int32(COLT)
    nchunk = (ROWS // RSUB) * (COLT // CSUB)

    def make_chunk(guarded):
        def chunk(k, _):
            ri = pl.multiple_of((k // (COLT // CSUB)) * RSUB, RSUB)
            ci = pl.multiple_of((k % (COLT // CSUB)) * CSUB, 256)
            m = mask_ref[pl.ds(ri, RSUB), pl.ds(ci, CSUB)]
            if guarded:
                valid = m & (lane < (jnp.int32(N_ACT) - c * jnp.int32(COLT)
                                     - ci))
            else:
                valid = m
            bits = _threefry_bits(base2d + (scal0 + ri * jnp.int32(N_ACT)
                                            + ci))
            key = (lax.shift_right_logical(bits, jnp.int32(9 - SB))
                   & jnp.int32(KEYMASK)) | revstrip
            v = jnp.where(valid, key, jnp.int32(-1))
            ka = key_acc[pl.ds(ri, RSUB), pl.ds(ci, CSUB)]
            key_acc[pl.ds(ri, RSUB), pl.ds(ci, CSUB)] = jnp.maximum(ka, v)
            ca = cnt_acc[pl.ds(ri, RSUB), pl.ds(ci, CSUB)]
            cnt_acc[pl.ds(ri, RSUB), pl.ds(ci, CSUB)] = \
                ca + valid.astype(jnp.int32)
            return 0
        return chunk

    @pl.when(c < CBLOCKS - 1)
    def _main():
        lax.fori_loop(0, nchunk, make_chunk(False), 0, unroll=4)

    @pl.when(c == CBLOCKS - 1)
    def _tail():
        lax.fori_loop(0, nchunk, make_chunk(True), 0, unroll=4)

    @pl.when(c == CBLOCKS - 1)
    def _fin():
        lane_f = lax.broadcasted_iota(jnp.int32, (RSUB, COLT), 1)
        for ri in range(ROWS // RSUB):
            keys = key_acc[pl.ds(ri * RSUB, RSUB), :]
            bb = lax.shift_right_arithmetic(keys, jnp.int32(SB))
            strip = jnp.int32(CBLOCKS - 1) - (keys & jnp.int32(2**SB - 1))
            gcol = strip * jnp.int32(COLT) + lane_f
            mx = jnp.max(bb, axis=1, keepdims=True)
            act_ref[pl.ds(ri * RSUB, RSUB), :] = jnp.min(
                jnp.where(bb == mx, gcol, jnp.int32(2**30)),
                axis=1, keepdims=True)
            cnt = jnp.sum(cnt_acc[pl.ds(ri * RSUB, RSUB), :],
                          axis=1, keepdims=True)
            lp_ref[pl.ds(ri * RSUB, RSUB), :] = -jnp.log(
                cnt.astype(jnp.float32))


@jax.jit
def _sample(mask):
    act, lp = pl.pallas_call(
        _body,
        grid=(BATCH // ROWS, CBLOCKS),
        in_specs=[pl.BlockSpec((ROWS, COLT), lambda r, c: (r, c))],
        out_specs=[pl.BlockSpec((ROWS, 1), lambda r, c: (r, 0)),
                   pl.BlockSpec((ROWS, 1), lambda r, c: (r, 0))],
        out_shape=[jax.ShapeDtypeStruct((BATCH, 1), jnp.int32),
                   jax.ShapeDtypeStruct((BATCH, 1), jnp.float32)],
        scratch_shapes=[pltpu.VMEM((ROWS, COLT), jnp.int32),
                        pltpu.VMEM((ROWS, COLT), jnp.int32)],
        compiler_params=pltpu.CompilerParams(
            dimension_semantics=("arbitrary", "arbitrary")),
    )(mask)
    return act[:, 0], lp[:, 0]


def kernel(action_mask, fc_w, fc_b):
    del fc_w, fc_b
    return _sample(action_mask.astype(jnp.bool_))
